# baseline (device time: 104961 ns/iter reference)
import jax
import jax.numpy as jnp
from jax import lax
from jax.experimental import pallas as pl
from jax.experimental.pallas import tpu as pltpu

N_DEV = 8
N_EXP = 32
E_PER = N_EXP // N_DEV
CAP = 25
N_TOK = 1024
D_IN = 256
D_OUT = 512


def kernel(x, router_W, route_idx, expert_W):
    del router_W

    def body(x_ref, ridx_ref, ew_ref, out_ref, comm_ref, send_sems, recv_sems):
        my_pos = lax.axis_index("i")
        left = lax.rem(my_pos + N_DEV - 1, N_DEV)
        right = lax.rem(my_pos + 1, N_DEV)

        barrier_sem = pltpu.get_barrier_semaphore()
        for nbr in (left, right):
            pl.semaphore_signal(
                barrier_sem, inc=1,
                device_id=(nbr,), device_id_type=pl.DeviceIdType.MESH,
            )
        pl.semaphore_wait(barrier_sem, 2)

        ridx = ridx_ref[:, :]
        e_ids = lax.broadcasted_iota(jnp.int32, (N_TOK, N_EXP), 1)
        onehot = ridx == e_ids
        row = lax.broadcasted_iota(jnp.int32, (N_TOK, N_TOK), 0)
        col = lax.broadcasted_iota(jnp.int32, (N_TOK, N_TOK), 1)
        tri = (row >= col).astype(jnp.bfloat16)
        counts_incl = jnp.dot(
            tri, onehot.astype(jnp.bfloat16),
            preferred_element_type=jnp.float32,
        )
        rank_incl = jnp.sum(
            jnp.where(onehot, counts_incl, 0.0), axis=1, keepdims=True
        )
        keep = rank_incl <= float(CAP)

        xb = x_ref[:, :].astype(jnp.bfloat16)
        acc = jnp.zeros((N_TOK, D_OUT), jnp.float32)
        for k in range(E_PER):
            e_k = my_pos * E_PER + k
            mask = jnp.logical_and(keep, ridx == e_k)
            xk = jnp.where(mask, xb, jnp.bfloat16(0.0))
            acc = acc + jnp.dot(
                xk, ew_ref[k].astype(jnp.bfloat16),
                preferred_element_type=jnp.float32,
            )

        out_ref[:, :] = acc
        comm_ref[0] = acc.astype(jnp.bfloat16)

        for h in range(N_DEV - 1):
            rdma = pltpu.make_async_remote_copy(
                src_ref=comm_ref.at[h],
                dst_ref=comm_ref.at[h + 1],
                send_sem=send_sems.at[h],
                recv_sem=recv_sems.at[h + 1],
                device_id=(right,),
                device_id_type=pl.DeviceIdType.MESH,
            )
            rdma.start()
            rdma.wait()
            out_ref[:, :] += comm_ref[h + 1].astype(jnp.float32)

    return pl.pallas_call(
        body,
        out_shape=jax.ShapeDtypeStruct((N_TOK, D_OUT), jnp.float32),
        in_specs=[
            pl.BlockSpec(memory_space=pltpu.VMEM),
            pl.BlockSpec(memory_space=pltpu.VMEM),
            pl.BlockSpec(memory_space=pltpu.VMEM),
        ],
        out_specs=pl.BlockSpec(memory_space=pltpu.VMEM),
        scratch_shapes=[
            pltpu.VMEM((N_DEV, N_TOK, D_OUT), jnp.bfloat16),
            pltpu.SemaphoreType.DMA((N_DEV,)),
            pltpu.SemaphoreType.DMA((N_DEV,)),
        ],
        compiler_params=pltpu.CompilerParams(collective_id=0),
    )(x, route_idx, expert_W)


# device time: 30102 ns/iter; 3.4868x vs baseline; 3.4868x over previous
import jax
import jax.numpy as jnp
from jax import lax
from jax.experimental import pallas as pl
from jax.experimental.pallas import tpu as pltpu

N_DEV = 8
N_EXP = 32
E_PER = N_EXP // N_DEV
CAP = 25
N_TOK = 1024
D_IN = 256
D_OUT = 512
CHUNK = N_TOK // N_DEV


def kernel(x, router_W, route_idx, expert_W):
    del router_W

    def body(
        x_ref, ridx_ref, ew_ref, out_ref,
        part_ref,
        rs_ref,
        ag_src_ref,
        ag_ref,
        rs_send_sems, rs_recv_sems, ag_send_sems, ag_recv_sems,
    ):
        my_pos = lax.axis_index("i")

        barrier_sem = pltpu.get_barrier_semaphore()
        for d in range(1, N_DEV):
            peer = lax.rem(my_pos + d, N_DEV)
            pl.semaphore_signal(
                barrier_sem, inc=1,
                device_id=(peer,), device_id_type=pl.DeviceIdType.MESH,
            )
        pl.semaphore_wait(barrier_sem, N_DEV - 1)

        ridx = ridx_ref[:, :]
        e_ids = lax.broadcasted_iota(jnp.int32, (N_TOK, N_EXP), 1)
        onehot = ridx == e_ids
        row = lax.broadcasted_iota(jnp.int32, (N_TOK, N_TOK), 0)
        col = lax.broadcasted_iota(jnp.int32, (N_TOK, N_TOK), 1)
        tri = (row >= col).astype(jnp.bfloat16)
        counts_incl = jnp.dot(
            tri, onehot.astype(jnp.bfloat16),
            preferred_element_type=jnp.float32,
        )
        rank_incl = jnp.sum(
            jnp.where(onehot, counts_incl, 0.0), axis=1, keepdims=True
        )
        keep = rank_incl <= float(CAP)

        xb = x_ref[:, :].astype(jnp.bfloat16)
        acc = jnp.zeros((N_TOK, D_OUT), jnp.float32)
        for k in range(E_PER):
            e_k = my_pos * E_PER + k
            mask = jnp.logical_and(keep, ridx == e_k)
            xk = jnp.where(mask, xb, jnp.bfloat16(0.0))
            acc = acc + jnp.dot(
                xk, ew_ref[k].astype(jnp.bfloat16),
                preferred_element_type=jnp.float32,
            )
        part_ref[...] = acc.astype(jnp.bfloat16).reshape(N_DEV, CHUNK, D_OUT)

        for d in range(1, N_DEV):
            q = lax.rem(my_pos + d, N_DEV)
            rdma = pltpu.make_async_remote_copy(
                src_ref=part_ref.at[q],
                dst_ref=rs_ref.at[N_DEV - d],
                send_sem=rs_send_sems.at[d],
                recv_sem=rs_recv_sems.at[N_DEV - d],
                device_id=(q,),
                device_id_type=pl.DeviceIdType.MESH,
            )
            rdma.start()

        red = part_ref[my_pos].astype(jnp.float32)
        for s in range(1, N_DEV):
            recv = pltpu.make_async_remote_copy(
                src_ref=rs_ref.at[s],
                dst_ref=rs_ref.at[s],
                send_sem=ag_send_sems.at[0],
                recv_sem=rs_recv_sems.at[s],
                device_id=(my_pos,),
                device_id_type=pl.DeviceIdType.MESH,
            )
            recv.wait_recv()
            red = red + rs_ref[s].astype(jnp.float32)
        ag_src_ref[...] = red.astype(jnp.bfloat16)

        for d in range(1, N_DEV):
            q = lax.rem(my_pos + d, N_DEV)
            rdma = pltpu.make_async_remote_copy(
                src_ref=ag_src_ref,
                dst_ref=ag_ref.at[N_DEV - d],
                send_sem=ag_send_sems.at[d],
                recv_sem=ag_recv_sems.at[N_DEV - d],
                device_id=(q,),
                device_id_type=pl.DeviceIdType.MESH,
            )
            rdma.start()

        out_ref[pl.ds(my_pos * CHUNK, CHUNK), :] = red
        for s in range(1, N_DEV):
            recv = pltpu.make_async_remote_copy(
                src_ref=ag_ref.at[s],
                dst_ref=ag_ref.at[s],
                send_sem=rs_send_sems.at[0],
                recv_sem=ag_recv_sems.at[s],
                device_id=(my_pos,),
                device_id_type=pl.DeviceIdType.MESH,
            )
            recv.wait_recv()
            c = lax.rem(my_pos + s, N_DEV)
            out_ref[pl.ds(c * CHUNK, CHUNK), :] = ag_ref[s].astype(jnp.float32)

        for d in range(1, N_DEV):
            for sems, src, dst in (
                (rs_send_sems, part_ref.at[0], rs_ref.at[d]),
                (ag_send_sems, ag_src_ref, ag_ref.at[d]),
            ):
                snd = pltpu.make_async_remote_copy(
                    src_ref=src,
                    dst_ref=dst,
                    send_sem=sems.at[d],
                    recv_sem=rs_recv_sems.at[0],
                    device_id=(my_pos,),
                    device_id_type=pl.DeviceIdType.MESH,
                )
                snd.wait_send()

    return pl.pallas_call(
        body,
        out_shape=jax.ShapeDtypeStruct((N_TOK, D_OUT), jnp.float32),
        in_specs=[
            pl.BlockSpec(memory_space=pltpu.VMEM),
            pl.BlockSpec(memory_space=pltpu.VMEM),
            pl.BlockSpec(memory_space=pltpu.VMEM),
        ],
        out_specs=pl.BlockSpec(memory_space=pltpu.VMEM),
        scratch_shapes=[
            pltpu.VMEM((N_DEV, CHUNK, D_OUT), jnp.bfloat16),
            pltpu.VMEM((N_DEV, CHUNK, D_OUT), jnp.bfloat16),
            pltpu.VMEM((CHUNK, D_OUT), jnp.bfloat16),
            pltpu.VMEM((N_DEV, CHUNK, D_OUT), jnp.bfloat16),
            pltpu.SemaphoreType.DMA((N_DEV,)),
            pltpu.SemaphoreType.DMA((N_DEV,)),
            pltpu.SemaphoreType.DMA((N_DEV,)),
            pltpu.SemaphoreType.DMA((N_DEV,)),
        ],
        compiler_params=pltpu.CompilerParams(collective_id=0),
    )(x, route_idx, expert_W)


# device time: 29396 ns/iter; 3.5706x vs baseline; 1.0240x over previous
import jax
import jax.numpy as jnp
from jax import lax
from jax.experimental import pallas as pl
from jax.experimental.pallas import tpu as pltpu

N_DEV = 8
N_EXP = 32
E_PER = N_EXP // N_DEV
CAP = 25
N_TOK = 1024
D_IN = 256
D_OUT = 512
CHUNK = N_TOK // N_DEV


def kernel(x, router_W, route_idx, expert_W):
    del router_W

    def body(
        x_ref, ridx_ref, ew_ref, out_ref,
        xk_ref,
        part_ref,
        rs_ref,
        ag_src_ref,
        ag_ref,
        rs_send_sems, rs_recv_sems, ag_send_sems, ag_recv_sems,
    ):
        my_pos = lax.axis_index("i")

        barrier_sem = pltpu.get_barrier_semaphore()
        for d in range(1, N_DEV):
            peer = lax.rem(my_pos + d, N_DEV)
            pl.semaphore_signal(
                barrier_sem, inc=1,
                device_id=(peer,), device_id_type=pl.DeviceIdType.MESH,
            )

        ridx = ridx_ref[:, :]
        e_ids = lax.broadcasted_iota(jnp.int32, (N_TOK, N_EXP), 1)
        onehot = ridx == e_ids
        row = lax.broadcasted_iota(jnp.int32, (N_TOK, N_TOK), 0)
        col = lax.broadcasted_iota(jnp.int32, (N_TOK, N_TOK), 1)
        tri = (row >= col).astype(jnp.bfloat16)
        counts_incl = jnp.dot(
            tri, onehot.astype(jnp.bfloat16),
            preferred_element_type=jnp.float32,
        )
        rank_incl = jnp.sum(
            jnp.where(onehot, counts_incl, 0.0), axis=1, keepdims=True
        )
        keep = rank_incl <= float(CAP)

        xb = x_ref[:, :].astype(jnp.bfloat16)
        ew = [ew_ref[k].astype(jnp.bfloat16) for k in range(E_PER)]
        for k in range(E_PER):
            e_k = my_pos * E_PER + k
            mask = jnp.logical_and(keep, ridx == e_k)
            xk_ref[k] = jnp.where(mask, xb, jnp.bfloat16(0.0))

        pl.semaphore_wait(barrier_sem, N_DEV - 1)

        for d in list(range(1, N_DEV)) + [0]:
            q = lax.rem(my_pos + d, N_DEV)
            r0 = q * CHUNK
            acc_q = jnp.zeros((CHUNK, D_OUT), jnp.float32)
            for k in range(E_PER):
                acc_q = acc_q + jnp.dot(
                    xk_ref[k, pl.ds(r0, CHUNK), :], ew[k],
                    preferred_element_type=jnp.float32,
                )
            part_ref[q] = acc_q.astype(jnp.bfloat16)
            if d > 0:
                rdma = pltpu.make_async_remote_copy(
                    src_ref=part_ref.at[q],
                    dst_ref=rs_ref.at[N_DEV - d],
                    send_sem=rs_send_sems.at[d],
                    recv_sem=rs_recv_sems.at[N_DEV - d],
                    device_id=(q,),
                    device_id_type=pl.DeviceIdType.MESH,
                )
                rdma.start()

        red = part_ref[my_pos].astype(jnp.float32)
        for s in range(1, N_DEV):
            recv = pltpu.make_async_remote_copy(
                src_ref=rs_ref.at[s],
                dst_ref=rs_ref.at[s],
                send_sem=ag_send_sems.at[0],
                recv_sem=rs_recv_sems.at[s],
                device_id=(my_pos,),
                device_id_type=pl.DeviceIdType.MESH,
            )
            recv.wait_recv()
            red = red + rs_ref[s].astype(jnp.float32)
        ag_src_ref[...] = red.astype(jnp.bfloat16)

        for d in range(1, N_DEV):
            q = lax.rem(my_pos + d, N_DEV)
            rdma = pltpu.make_async_remote_copy(
                src_ref=ag_src_ref,
                dst_ref=ag_ref.at[N_DEV - d],
                send_sem=ag_send_sems.at[d],
                recv_sem=ag_recv_sems.at[N_DEV - d],
                device_id=(q,),
                device_id_type=pl.DeviceIdType.MESH,
            )
            rdma.start()

        out_ref[pl.ds(my_pos * CHUNK, CHUNK), :] = red
        for s in range(1, N_DEV):
            recv = pltpu.make_async_remote_copy(
                src_ref=ag_ref.at[s],
                dst_ref=ag_ref.at[s],
                send_sem=rs_send_sems.at[0],
                recv_sem=ag_recv_sems.at[s],
                device_id=(my_pos,),
                device_id_type=pl.DeviceIdType.MESH,
            )
            recv.wait_recv()
            c = lax.rem(my_pos + s, N_DEV)
            out_ref[pl.ds(c * CHUNK, CHUNK), :] = ag_ref[s].astype(jnp.float32)

        for d in range(1, N_DEV):
            for sems, src, dst in (
                (rs_send_sems, part_ref.at[0], rs_ref.at[d]),
                (ag_send_sems, ag_src_ref, ag_ref.at[d]),
            ):
                snd = pltpu.make_async_remote_copy(
                    src_ref=src,
                    dst_ref=dst,
                    send_sem=sems.at[d],
                    recv_sem=rs_recv_sems.at[0],
                    device_id=(my_pos,),
                    device_id_type=pl.DeviceIdType.MESH,
                )
                snd.wait_send()

    return pl.pallas_call(
        body,
        out_shape=jax.ShapeDtypeStruct((N_TOK, D_OUT), jnp.float32),
        in_specs=[
            pl.BlockSpec(memory_space=pltpu.VMEM),
            pl.BlockSpec(memory_space=pltpu.VMEM),
            pl.BlockSpec(memory_space=pltpu.VMEM),
        ],
        out_specs=pl.BlockSpec(memory_space=pltpu.VMEM),
        scratch_shapes=[
            pltpu.VMEM((E_PER, N_TOK, D_IN), jnp.bfloat16),
            pltpu.VMEM((N_DEV, CHUNK, D_OUT), jnp.bfloat16),
            pltpu.VMEM((N_DEV, CHUNK, D_OUT), jnp.bfloat16),
            pltpu.VMEM((CHUNK, D_OUT), jnp.bfloat16),
            pltpu.VMEM((N_DEV, CHUNK, D_OUT), jnp.bfloat16),
            pltpu.SemaphoreType.DMA((N_DEV,)),
            pltpu.SemaphoreType.DMA((N_DEV,)),
            pltpu.SemaphoreType.DMA((N_DEV,)),
            pltpu.SemaphoreType.DMA((N_DEV,)),
        ],
        compiler_params=pltpu.CompilerParams(collective_id=0),
    )(x, route_idx, expert_W)


# device time: 18993 ns/iter; 5.5263x vs baseline; 1.5477x over previous
import jax
import jax.numpy as jnp
from jax import lax
from jax.experimental import pallas as pl
from jax.experimental.pallas import tpu as pltpu

N_DEV = 8
N_EXP = 32
E_PER = N_EXP // N_DEV
CAP = 25
N_TOK = 1024
D_IN = 256
D_OUT = 512
R_PER_E = 26
C_ROWS = E_PER * R_PER_E
BLK = 128
N_BLK = N_TOK // BLK
MSG_ROWS = C_ROWS + 8


def kernel(x, router_W, route_idx, expert_W):
    del router_W

    def body(
        x_ref, ridx_ref, ew_ref, out_ref,
        gt_ref,
        msg_src_ref,
        msg_ref,
        send_sems, recv_sems,
    ):
        my_pos = lax.axis_index("i")

        barrier_sem = pltpu.get_barrier_semaphore()
        for d in range(1, N_DEV):
            peer = lax.rem(my_pos + d, N_DEV)
            pl.semaphore_signal(
                barrier_sem, inc=1,
                device_id=(peer,), device_id_type=pl.DeviceIdType.MESH,
            )

        ridx = ridx_ref[:, :]
        e_ids = lax.broadcasted_iota(jnp.int32, (N_TOK, N_EXP), 1)
        oh = (ridx == e_ids)
        oh_b = oh.astype(jnp.bfloat16)
        row = lax.broadcasted_iota(jnp.int32, (BLK, BLK), 0)
        col = lax.broadcasted_iota(jnp.int32, (BLK, BLK), 1)
        tri = (row >= col).astype(jnp.bfloat16)
        r_ids = lax.broadcasted_iota(jnp.int32, (1, R_PER_E), 1).astype(
            jnp.float32
        )

        prefix = jnp.zeros((1, N_EXP), jnp.float32)
        for n in range(N_BLK):
            r0, r1 = n * BLK, (n + 1) * BLK
            oh_blk = oh_b[r0:r1, :]
            cnt = jnp.dot(
                tri, oh_blk, preferred_element_type=jnp.float32
            ) + prefix
            rank = jnp.sum(
                jnp.where(oh[r0:r1, :], cnt, 0.0), axis=1, keepdims=True
            )
            keep_blk = rank <= float(CAP)
            ridx_blk = ridx[r0:r1, :]
            for k in range(E_PER):
                e_k = my_pos * E_PER + k
                m = jnp.logical_and(keep_blk, ridx_blk == e_k)
                onehot_rank = jnp.logical_and(m, rank == r_ids + 1.0)
                gt_ref[r0:r1, k * R_PER_E:(k + 1) * R_PER_E] = (
                    onehot_rank.astype(jnp.bfloat16)
                )
            prefix = prefix + jnp.sum(oh_blk, axis=0, keepdims=True).astype(
                jnp.float32
            )

        gtv = gt_ref[...]
        xb = x_ref[:, :].astype(jnp.bfloat16)
        xg = lax.dot_general(
            gtv, xb, (((0,), (0,)), ((), ())),
            preferred_element_type=jnp.float32,
        ).astype(jnp.bfloat16)
        cks = []
        for k in range(E_PER):
            ck = jnp.dot(
                xg[k * R_PER_E:(k + 1) * R_PER_E, :],
                ew_ref[k].astype(jnp.bfloat16),
                preferred_element_type=jnp.float32,
            )
            cks.append(ck.astype(jnp.bfloat16))
        tok_mat = lax.broadcasted_iota(jnp.int32, (N_TOK, C_ROWS), 0).astype(
            jnp.float32
        )
        idxrow = jnp.sum(
            gtv.astype(jnp.float32) * tok_mat, axis=0, keepdims=True
        )
        hi = jnp.floor(idxrow / 64.0)
        lo = idxrow - 64.0 * hi
        lane_pad = jnp.zeros((1, D_OUT - C_ROWS), jnp.float32)
        row_hi = jnp.concatenate([hi, lane_pad], axis=1).astype(jnp.bfloat16)
        row_lo = jnp.concatenate([lo, lane_pad], axis=1).astype(jnp.bfloat16)
        pad = jnp.zeros((MSG_ROWS - C_ROWS - 2, D_OUT), jnp.bfloat16)
        msg_src_ref[...] = jnp.concatenate(cks + [row_hi, row_lo, pad], axis=0)

        pl.semaphore_wait(barrier_sem, N_DEV - 1)

        for d in range(1, N_DEV):
            q = lax.rem(my_pos + d, N_DEV)
            rdma = pltpu.make_async_remote_copy(
                src_ref=msg_src_ref,
                dst_ref=msg_ref.at[N_DEV - d],
                send_sem=send_sems.at[d],
                recv_sem=recv_sems.at[N_DEV - d],
                device_id=(q,),
                device_id_type=pl.DeviceIdType.MESH,
            )
            rdma.start()

        tok_col = lax.broadcasted_iota(jnp.int32, (N_TOK, 1), 0).astype(jnp.float32)

        def scatter(idx_vals, comp_vals, acc):
            p_mat = (tok_col == idx_vals).astype(jnp.bfloat16)
            return acc + jnp.dot(
                p_mat, comp_vals, preferred_element_type=jnp.float32
            )

        def unpack_scatter(blk, acc):
            comp_vals = blk[0:C_ROWS, :]
            hi_r = blk[C_ROWS:C_ROWS + 1, 0:C_ROWS].astype(jnp.float32)
            lo_r = blk[C_ROWS + 1:C_ROWS + 2, 0:C_ROWS].astype(jnp.float32)
            return scatter(hi_r * 64.0 + lo_r, comp_vals, acc)

        acc = scatter(idxrow, jnp.concatenate(cks, axis=0), jnp.zeros(
            (N_TOK, D_OUT), jnp.float32
        ))
        for s in range(N_DEV - 1, 0, -1):
            recv = pltpu.make_async_remote_copy(
                src_ref=msg_ref.at[s],
                dst_ref=msg_ref.at[s],
                send_sem=send_sems.at[0],
                recv_sem=recv_sems.at[s],
                device_id=(my_pos,),
                device_id_type=pl.DeviceIdType.MESH,
            )
            recv.wait_recv()
            acc = unpack_scatter(msg_ref[s], acc)
        out_ref[...] = acc

        for d in range(1, N_DEV):
            snd = pltpu.make_async_remote_copy(
                src_ref=msg_src_ref,
                dst_ref=msg_ref.at[d],
                send_sem=send_sems.at[d],
                recv_sem=recv_sems.at[0],
                device_id=(my_pos,),
                device_id_type=pl.DeviceIdType.MESH,
            )
            snd.wait_send()

    return pl.pallas_call(
        body,
        out_shape=jax.ShapeDtypeStruct((N_TOK, D_OUT), jnp.float32),
        in_specs=[
            pl.BlockSpec(memory_space=pltpu.VMEM),
            pl.BlockSpec(memory_space=pltpu.VMEM),
            pl.BlockSpec(memory_space=pltpu.VMEM),
        ],
        out_specs=pl.BlockSpec(memory_space=pltpu.VMEM),
        scratch_shapes=[
            pltpu.VMEM((N_TOK, C_ROWS), jnp.bfloat16),
            pltpu.VMEM((MSG_ROWS, D_OUT), jnp.bfloat16),
            pltpu.VMEM((N_DEV, MSG_ROWS, D_OUT), jnp.bfloat16),
            pltpu.SemaphoreType.DMA((N_DEV,)),
            pltpu.SemaphoreType.DMA((N_DEV,)),
        ],
        compiler_params=pltpu.CompilerParams(collective_id=0),
    )(x, route_idx, expert_W)
